# baseline (speedup 1.0000x reference)
import jax
import jax.numpy as jnp
from jax import lax
from jax.experimental import pallas as pl
from jax.experimental.pallas import tpu as pltpu
from jax.experimental.pallas import tpu_sc as plsc

N = 10000
D = 128
DH = D // 2
NNZ = 320000

NC = 2
NS = 16
NW = NC * NS
CHUNK = 64
STEPS = 320
PPT = STEPS * CHUNK
NNZP = NS * PPT
NP = 10240
RPT = NP // NS

_MESH = plsc.VectorSubcoreMesh(
    core_axis_name="c", subcore_axis_name="s", num_cores=NC, num_subcores=NS
)
_SC_PARAMS = pltpu.CompilerParams(
    needs_layout_passes=False, use_tc_tiling_on_sc=False
)


_RB = 1024


def _proj_matmul_body(y_ref, th_ref, o_ref):
    y = y_ref[...]
    norm = jnp.sqrt(jnp.sum(y * y, axis=-1, keepdims=True))
    yp = y / jnp.maximum(norm, 1e-7)
    o_ref[0] = jnp.dot(yp, th_ref[0], preferred_element_type=jnp.float32)


def _proj_matmul(y_pad, theta_split):
    return pl.pallas_call(
        _proj_matmul_body,
        grid=(NC, NP // _RB),
        in_specs=[
            pl.BlockSpec((_RB, D), lambda c, i: (i, 0)),
            pl.BlockSpec((1, D, DH), lambda c, i: (c, 0, 0)),
        ],
        out_specs=pl.BlockSpec((1, _RB, DH), lambda c, i: (c, i, 0)),
        out_shape=jax.ShapeDtypeStruct((NC, NP, DH), jnp.float32),
    )(y_pad, theta_split)


def _combine_div_body(part_ref, deg_ref, o_ref):
    deg = jnp.sum(deg_ref[...], axis=0)
    inv = 1.0 / jnp.maximum(deg, 1.0)
    x = jnp.concatenate([part_ref[0], part_ref[1]], axis=-1)
    o_ref[...] = x * inv[:, None]


def _combine_div(part, deg_part):
    return pl.pallas_call(
        _combine_div_body,
        grid=(NP // _RB,),
        in_specs=[
            pl.BlockSpec((NC, _RB, DH), lambda i: (0, i, 0)),
            pl.BlockSpec((NS, _RB), lambda i: (0, i)),
        ],
        out_specs=pl.BlockSpec((_RB, D), lambda i: (i, 0)),
        out_shape=jax.ShapeDtypeStruct((NP, D), jnp.float32),
    )(part, deg_part)


def _finish_body(part_ref, deg_ref, b_ref, o_ref):
    deg = jnp.sum(deg_ref[...], axis=0)
    inv = 1.0 / jnp.maximum(deg, 1.0)
    x = part_ref[0] + part_ref[1]
    o_ref[...] = jnp.tanh(x * inv[:, None] + b_ref[...])


def _finish(part, deg_part, bias):
    return pl.pallas_call(
        _finish_body,
        grid=(NP // _RB,),
        in_specs=[
            pl.BlockSpec((NC, _RB, D), lambda i: (0, i, 0)),
            pl.BlockSpec((NS, _RB), lambda i: (0, i)),
            pl.BlockSpec((D,), lambda i: (0,)),
        ],
        out_specs=pl.BlockSpec((_RB, D), lambda i: (i, 0)),
        out_shape=jax.ShapeDtypeStruct((NP, D), jnp.float32),
    )(part, deg_part, bias)


PK = 4
_DIAG_SCATTER = True


def _pipelined_gather_scatter(table_c, gidx, sidx, rowbuf, acc,
                              gsA, gsB, ssA, ssB, steps, chunk, pk):
    def a_buf(i):
        return rowbuf.at[i]

    def b_buf(i):
        return rowbuf.at[pk + i]

    def wait_gather(buf, sem):
        pltpu.make_async_copy(table_c.at[pl.ds(0, chunk)], buf, sem).wait()

    def wait_scatter(buf, sem):
        pltpu.make_async_copy(buf, acc.at[sidx.at[0]], sem).wait()

    def body(t, carry):
        base = t * 2 * pk

        if _DIAG_SCATTER:
            @pl.when(t > 0)
            def _():
                for i in range(pk):
                    wait_scatter(b_buf(i), ssB)

        for i in range(pk):
            pltpu.async_copy(table_c.at[gidx.at[base + pk + i]],
                             b_buf(i), gsB)
        for i in range(pk):
            wait_gather(a_buf(i), gsA)
        if _DIAG_SCATTER:
            for i in range(pk):
                pltpu.async_copy(a_buf(i), acc.at[sidx.at[base + i]],
                                 ssA, add=True)
            for i in range(pk):
                wait_scatter(a_buf(i), ssA)

        @pl.when(base + 2 * pk < steps)
        def _():
            for i in range(pk):
                pltpu.async_copy(table_c.at[gidx.at[base + 2 * pk + i]],
                                 a_buf(i), gsA)

        for i in range(pk):
            wait_gather(b_buf(i), gsB)
        if _DIAG_SCATTER:
            for i in range(pk):
                pltpu.async_copy(b_buf(i), acc.at[sidx.at[base + pk + i]],
                                 ssB, add=True)
        return carry

    lax.fori_loop(0, steps // (2 * pk), body, 0)
    if _DIAG_SCATTER:
        for i in range(pk):
            wait_scatter(b_buf(i), ssB)


def _sc_stage1_body(
    z_hbm, n3_hbm, e3_hbm, zeros_hbm,
    sum_hbm, degv_hbm, dege_hbm,
    gidx, sidx, rowbuf, deg_v, acc, gsA, gsB, ssA, ssB, isem,
):
    c = lax.axis_index("c")
    s = lax.axis_index("s")
    r0 = s * RPT
    table_c = z_hbm.at[c]

    pltpu.async_copy(n3_hbm.at[s], gidx, isem)
    pltpu.async_copy(e3_hbm.at[s], sidx, isem)
    pltpu.make_async_copy(n3_hbm.at[s], gidx, isem).wait()
    pltpu.make_async_copy(e3_hbm.at[s], sidx, isem).wait()

    for i in range(PK):
        pltpu.async_copy(table_c.at[gidx.at[i]], rowbuf.at[i], gsA)

    pltpu.sync_copy(zeros_hbm.at[pl.ds(r0, RPT)], acc.at[pl.ds(r0, RPT)])

    z16 = jnp.zeros((16,), jnp.float32)

    def zero_body(i, carry):
        deg_v[pl.ds(i * 16, 16)] = z16
        return carry

    lax.fori_loop(0, NP // 16, zero_body, 0)

    ones16 = jnp.ones((16,), jnp.float32)

    def make_deg_body(src):
        def deg_body(j, carry):
            for k in range(CHUNK // 16):
                iv = src[j, pl.ds(k * 16, 16)]
                plsc.addupdate_scatter(deg_v, [iv], ones16)
            return carry
        return deg_body

    @pl.when(c == 0)
    def _():
        lax.fori_loop(0, STEPS, make_deg_body(gidx), 0)

    @pl.when(c != 0)
    def _():
        lax.fori_loop(0, STEPS, make_deg_body(sidx), 0)

    plsc.subcore_barrier()

    _pipelined_gather_scatter(table_c, gidx, sidx, rowbuf, acc,
                              gsA, gsB, ssA, ssB, STEPS, CHUNK, PK)
    plsc.subcore_barrier()

    pltpu.sync_copy(acc.at[pl.ds(r0, RPT)], sum_hbm.at[c, pl.ds(r0, RPT)])

    @pl.when(c == 0)
    def _():
        pltpu.sync_copy(deg_v, degv_hbm.at[s])

    @pl.when(c != 0)
    def _():
        pltpu.sync_copy(deg_v, dege_hbm.at[s])


C2 = 32
S2 = 320
PK2 = 2
PPW2 = S2 * C2


def _sc_stage2_body(
    ef_hbm, e3_hbm, n3_hbm, zeros_hbm,
    sum_hbm,
    gidx, sidx, rowbuf, acc, gsA, gsB, ssA, ssB, isem,
):
    c = lax.axis_index("c")
    s = lax.axis_index("s")
    wid = s * NC + c
    r0 = s * RPT

    pltpu.async_copy(e3_hbm.at[wid], gidx, isem)
    pltpu.async_copy(n3_hbm.at[wid], sidx, isem)
    pltpu.make_async_copy(e3_hbm.at[wid], gidx, isem).wait()
    pltpu.make_async_copy(n3_hbm.at[wid], sidx, isem).wait()

    for i in range(PK2):
        pltpu.async_copy(ef_hbm.at[gidx.at[i]], rowbuf.at[i], gsA)

    pltpu.sync_copy(zeros_hbm.at[pl.ds(r0, RPT)], acc.at[pl.ds(r0, RPT)])
    plsc.subcore_barrier()

    _pipelined_gather_scatter(ef_hbm, gidx, sidx, rowbuf, acc,
                              gsA, gsB, ssA, ssB, S2, C2, PK2)
    plsc.subcore_barrier()

    pltpu.sync_copy(acc.at[pl.ds(r0, RPT)], sum_hbm.at[c, pl.ds(r0, RPT)])


_sc_stage1 = pl.kernel(
    _sc_stage1_body,
    out_type=(
        jax.ShapeDtypeStruct((NC, NP, DH), jnp.float32),
        jax.ShapeDtypeStruct((NS, NP), jnp.float32),
        jax.ShapeDtypeStruct((NS, NP), jnp.float32),
    ),
    mesh=_MESH,
    compiler_params=_SC_PARAMS,
    scratch_types=[
        pltpu.VMEM((STEPS, CHUNK), jnp.int32),
        pltpu.VMEM((STEPS, CHUNK), jnp.int32),
        pltpu.VMEM((2 * PK, CHUNK, DH), jnp.float32),
        pltpu.VMEM((NP,), jnp.float32),
        pltpu.VMEM_SHARED((NP, DH), jnp.float32),
        pltpu.SemaphoreType.DMA,
        pltpu.SemaphoreType.DMA,
        pltpu.SemaphoreType.DMA,
        pltpu.SemaphoreType.DMA,
        pltpu.SemaphoreType.DMA,
    ],
)

_sc_stage2 = pl.kernel(
    _sc_stage2_body,
    out_type=jax.ShapeDtypeStruct((NC, NP, D), jnp.float32),
    mesh=_MESH,
    compiler_params=_SC_PARAMS,
    scratch_types=[
        pltpu.VMEM((S2, C2), jnp.int32),
        pltpu.VMEM((S2, C2), jnp.int32),
        pltpu.VMEM((2 * PK2, C2, D), jnp.float32),
        pltpu.VMEM_SHARED((NP, D), jnp.float32),
        pltpu.SemaphoreType.DMA,
        pltpu.SemaphoreType.DMA,
        pltpu.SemaphoreType.DMA,
        pltpu.SemaphoreType.DMA,
        pltpu.SemaphoreType.DMA,
    ],
)


def kernel(t, y, incidence, theta, bias):
    del t
    node_idx = incidence[0]
    edge_idx = incidence[1]
    pad = jnp.full((NNZP - NNZ,), N, dtype=jnp.int32)
    node_pad = jnp.concatenate([node_idx, pad])
    edge_pad = jnp.concatenate([edge_idx, pad])
    n3 = node_pad.reshape(NS, STEPS, CHUNK)
    e3 = edge_pad.reshape(NS, STEPS, CHUNK)
    n3w = node_pad.reshape(NW, S2, C2)
    e3w = edge_pad.reshape(NW, S2, C2)
    y_pad = jnp.concatenate(
        [y, jnp.zeros((NP - N, D), dtype=jnp.float32)], axis=0
    )
    zeros_h = jnp.zeros((NP, DH), dtype=jnp.float32)
    zeros_f = jnp.zeros((NP, D), dtype=jnp.float32)
    theta_split = jnp.stack([theta[:, :DH], theta[:, DH:]])

    z = _proj_matmul(y_pad, theta_split)
    esum, degv_p, dege_p = _sc_stage1(z, n3, e3, zeros_h)
    edge_feat = _combine_div(esum, dege_p)
    nsum = _sc_stage2(edge_feat, e3w, n3w, zeros_f)
    out = _finish(nsum, degv_p, bias)
    return out[:N]


# speedup vs baseline: 1.3466x; 1.3466x over previous
import jax
import jax.numpy as jnp
from jax import lax
from jax.experimental import pallas as pl
from jax.experimental.pallas import tpu as pltpu
from jax.experimental.pallas import tpu_sc as plsc

N = 10000
D = 128
DH = D // 2
NNZ = 320000

NC = 2
NS = 16
CH = 64
ST = 320
PK1 = 4
PK2 = 5
PPT = ST * CH
NNZP = NS * PPT
NP = 10240
RPT = NP // NS

_MESH = plsc.VectorSubcoreMesh(
    core_axis_name="c", subcore_axis_name="s", num_cores=NC, num_subcores=NS
)
_SC_PARAMS = pltpu.CompilerParams(
    needs_layout_passes=False, use_tc_tiling_on_sc=False
)


_RB = 1024


def _proj_matmul_body(y_ref, th_ref, o_ref):
    y = y_ref[...]
    norm = jnp.sqrt(jnp.sum(y * y, axis=-1, keepdims=True))
    yp = y / jnp.maximum(norm, 1e-7)
    o_ref[0] = jnp.dot(yp, th_ref[0], preferred_element_type=jnp.float32)


def _proj_matmul(y_pad, theta_split):
    return pl.pallas_call(
        _proj_matmul_body,
        grid=(NC, NP // _RB),
        in_specs=[
            pl.BlockSpec((_RB, D), lambda c, i: (i, 0)),
            pl.BlockSpec((1, D, DH), lambda c, i: (c, 0, 0)),
        ],
        out_specs=pl.BlockSpec((1, _RB, DH), lambda c, i: (c, i, 0)),
        out_shape=jax.ShapeDtypeStruct((NC, NP, DH), jnp.float32),
    )(y_pad, theta_split)


def _combine_div_body(part_ref, deg_ref, o_ref):
    deg = jnp.sum(deg_ref[...], axis=0)
    inv = 1.0 / jnp.maximum(deg, 1.0)
    o_ref[0] = part_ref[0] * inv[:, None]


def _combine_div(part, deg_part):
    return pl.pallas_call(
        _combine_div_body,
        grid=(NC, NP // _RB),
        in_specs=[
            pl.BlockSpec((1, _RB, DH), lambda c, i: (c, i, 0)),
            pl.BlockSpec((NS, _RB), lambda c, i: (0, i)),
        ],
        out_specs=pl.BlockSpec((1, _RB, DH), lambda c, i: (c, i, 0)),
        out_shape=jax.ShapeDtypeStruct((NC, NP, DH), jnp.float32),
    )(part, deg_part)


def _finish_body(part_ref, deg_ref, b_ref, o_ref):
    deg = jnp.sum(deg_ref[...], axis=0)
    inv = 1.0 / jnp.maximum(deg, 1.0)
    x = jnp.concatenate([part_ref[0], part_ref[1]], axis=-1)
    o_ref[...] = jnp.tanh(x * inv[:, None] + b_ref[...])


def _finish(part, deg_part, bias):
    return pl.pallas_call(
        _finish_body,
        grid=(NP // _RB,),
        in_specs=[
            pl.BlockSpec((NC, _RB, DH), lambda i: (0, i, 0)),
            pl.BlockSpec((NS, _RB), lambda i: (0, i)),
            pl.BlockSpec((D,), lambda i: (0,)),
        ],
        out_specs=pl.BlockSpec((_RB, D), lambda i: (i, 0)),
        out_shape=jax.ShapeDtypeStruct((NP, D), jnp.float32),
    )(part, deg_part, bias)


def _pipelined_gather_scatter(table_c, gidx, sidx, rowbuf, acc,
                              gsA, gsB, ssA, ssB, pk, hist_rows=None):
    def a_buf(i):
        return rowbuf.at[i]

    def b_buf(i):
        return rowbuf.at[pk + i]

    def wait_gather(buf, sem):
        pltpu.make_async_copy(table_c.at[pl.ds(0, CH)], buf, sem).wait()

    def wait_scatter(buf, sem):
        pltpu.make_async_copy(buf, acc.at[sidx.at[0]], sem).wait()

    def body(t, carry):
        base = t * 2 * pk

        @pl.when(t > 0)
        def _():
            for i in range(pk):
                wait_scatter(b_buf(i), ssB)

        for i in range(pk):
            pltpu.async_copy(table_c.at[gidx.at[base + pk + i]],
                             b_buf(i), gsB)

        if hist_rows is not None:
            hist_rows(base)

        for i in range(pk):
            wait_gather(a_buf(i), gsA)
        for i in range(pk):
            pltpu.async_copy(a_buf(i), acc.at[sidx.at[base + i]],
                             ssA, add=True)
        for i in range(pk):
            wait_scatter(a_buf(i), ssA)

        @pl.when(base + 2 * pk < ST)
        def _():
            for i in range(pk):
                pltpu.async_copy(table_c.at[gidx.at[base + 2 * pk + i]],
                                 a_buf(i), gsA)

        for i in range(pk):
            wait_gather(b_buf(i), gsB)
        for i in range(pk):
            pltpu.async_copy(b_buf(i), acc.at[sidx.at[base + pk + i]],
                             ssB, add=True)
        return carry

    lax.fori_loop(0, ST // (2 * pk), body, 0)
    for i in range(pk):
        wait_scatter(b_buf(i), ssB)


def _sc_stage1_body(
    z_hbm, n3_hbm, e3_hbm, zeros_hbm,
    sum_hbm, degv_hbm, dege_hbm,
    gidx, sidx, rowbuf, deg_h, acc, gsA, gsB, ssA, ssB, isem,
):
    c = lax.axis_index("c")
    s = lax.axis_index("s")
    r0 = s * RPT
    table_c = z_hbm.at[c]

    pltpu.async_copy(n3_hbm.at[s], gidx, isem)
    pltpu.async_copy(e3_hbm.at[s], sidx, isem)
    pltpu.make_async_copy(n3_hbm.at[s], gidx, isem).wait()
    pltpu.make_async_copy(e3_hbm.at[s], sidx, isem).wait()

    for i in range(PK1):
        pltpu.async_copy(table_c.at[gidx.at[i]], rowbuf.at[i], gsA)

    pltpu.sync_copy(zeros_hbm.at[pl.ds(r0, RPT)], acc.at[pl.ds(r0, RPT)])

    z16 = jnp.zeros((16,), jnp.float32)

    def zero_body(i, carry):
        deg_h[pl.ds(i * 16, 16)] = z16
        return carry

    lax.fori_loop(0, NP // 16, zero_body, 0)
    plsc.subcore_barrier()

    ones16 = jnp.ones((16,), jnp.float32)

    def make_hist(src):
        def hist_rows(base):
            for r in range(2 * PK1):
                for k in range(CH // 16):
                    iv = src[base + r, pl.ds(k * 16, 16)]
                    plsc.addupdate_scatter(deg_h, [iv], ones16)
        return hist_rows

    @pl.when(c == 0)
    def _():
        _pipelined_gather_scatter(table_c, gidx, sidx, rowbuf, acc,
                                  gsA, gsB, ssA, ssB, PK1,
                                  hist_rows=make_hist(gidx))

    @pl.when(c != 0)
    def _():
        _pipelined_gather_scatter(table_c, gidx, sidx, rowbuf, acc,
                                  gsA, gsB, ssA, ssB, PK1,
                                  hist_rows=make_hist(sidx))

    plsc.subcore_barrier()

    pltpu.sync_copy(acc.at[pl.ds(r0, RPT)], sum_hbm.at[c, pl.ds(r0, RPT)])

    @pl.when(c == 0)
    def _():
        pltpu.sync_copy(deg_h, degv_hbm.at[s])

    @pl.when(c != 0)
    def _():
        pltpu.sync_copy(deg_h, dege_hbm.at[s])


def _sc_stage2_body(
    ef_hbm, e3_hbm, n3_hbm, zeros_hbm,
    sum_hbm,
    gidx, sidx, rowbuf, acc, gsA, gsB, ssA, ssB, isem,
):
    c = lax.axis_index("c")
    s = lax.axis_index("s")
    r0 = s * RPT
    table_c = ef_hbm.at[c]

    pltpu.async_copy(e3_hbm.at[s], gidx, isem)
    pltpu.async_copy(n3_hbm.at[s], sidx, isem)
    pltpu.make_async_copy(e3_hbm.at[s], gidx, isem).wait()
    pltpu.make_async_copy(n3_hbm.at[s], sidx, isem).wait()

    for i in range(PK2):
        pltpu.async_copy(table_c.at[gidx.at[i]], rowbuf.at[i], gsA)

    pltpu.sync_copy(zeros_hbm.at[pl.ds(r0, RPT)], acc.at[pl.ds(r0, RPT)])
    plsc.subcore_barrier()

    _pipelined_gather_scatter(table_c, gidx, sidx, rowbuf, acc,
                              gsA, gsB, ssA, ssB, PK2)
    plsc.subcore_barrier()

    pltpu.sync_copy(acc.at[pl.ds(r0, RPT)], sum_hbm.at[c, pl.ds(r0, RPT)])


_sc_stage1 = pl.kernel(
    _sc_stage1_body,
    out_type=(
        jax.ShapeDtypeStruct((NC, NP, DH), jnp.float32),
        jax.ShapeDtypeStruct((NS, NP), jnp.float32),
        jax.ShapeDtypeStruct((NS, NP), jnp.float32),
    ),
    mesh=_MESH,
    compiler_params=_SC_PARAMS,
    scratch_types=[
        pltpu.VMEM((ST, CH), jnp.int32),
        pltpu.VMEM((ST, CH), jnp.int32),
        pltpu.VMEM((2 * PK1, CH, DH), jnp.float32),
        pltpu.VMEM((NP,), jnp.float32),
        pltpu.VMEM_SHARED((NP, DH), jnp.float32),
        pltpu.SemaphoreType.DMA,
        pltpu.SemaphoreType.DMA,
        pltpu.SemaphoreType.DMA,
        pltpu.SemaphoreType.DMA,
        pltpu.SemaphoreType.DMA,
    ],
)

_sc_stage2 = pl.kernel(
    _sc_stage2_body,
    out_type=jax.ShapeDtypeStruct((NC, NP, DH), jnp.float32),
    mesh=_MESH,
    compiler_params=_SC_PARAMS,
    scratch_types=[
        pltpu.VMEM((ST, CH), jnp.int32),
        pltpu.VMEM((ST, CH), jnp.int32),
        pltpu.VMEM((2 * PK2, CH, DH), jnp.float32),
        pltpu.VMEM_SHARED((NP, DH), jnp.float32),
        pltpu.SemaphoreType.DMA,
        pltpu.SemaphoreType.DMA,
        pltpu.SemaphoreType.DMA,
        pltpu.SemaphoreType.DMA,
        pltpu.SemaphoreType.DMA,
    ],
)


def kernel(t, y, incidence, theta, bias):
    del t
    node_idx = incidence[0]
    edge_idx = incidence[1]
    pad = jnp.full((NNZP - NNZ,), N, dtype=jnp.int32)
    n3 = jnp.concatenate([node_idx, pad]).reshape(NS, ST, CH)
    e3 = jnp.concatenate([edge_idx, pad]).reshape(NS, ST, CH)
    y_pad = jnp.concatenate(
        [y, jnp.zeros((NP - N, D), dtype=jnp.float32)], axis=0
    )
    zeros_h = jnp.zeros((NP, DH), dtype=jnp.float32)
    theta_split = jnp.stack([theta[:, :DH], theta[:, DH:]])

    z = _proj_matmul(y_pad, theta_split)
    esum, degv_p, dege_p = _sc_stage1(z, n3, e3, zeros_h)
    edge_feat = _combine_div(esum, dege_p)
    nsum = _sc_stage2(edge_feat, e3, n3, zeros_h)
    out = _finish(nsum, degv_p, bias)
    return out[:N]
